# baseline (speedup 1.0000x reference)
import jax
import jax.numpy as jnp
from jax import lax
from jax.experimental import pallas as pl
from jax.experimental.pallas import tpu as pltpu
from jax.experimental.pallas import tpu_sc as plsc

_NC = 2
_NS = 16
_L = 16

def _pre_body(x_ref, wa_ref, wb_ref, wp_ref, wq_ref, bb_ref, bp_ref, bq_ref,
              a_out, b_out, p_out, q_out):
    x = x_ref[...]
    a_out[...] = jnp.dot(x, wa_ref[...], preferred_element_type=jnp.float32)
    b_out[...] = jnp.dot(x, wb_ref[...], preferred_element_type=jnp.float32) + bb_ref[...]
    p_out[...] = jnp.dot(x, wp_ref[...], preferred_element_type=jnp.float32) + bp_ref[...]
    q_out[...] = jnp.dot(x, wq_ref[...], preferred_element_type=jnp.float32) + bq_ref[...]


def _post_body(x_ref, hin0_ref, hin1_ref, hout0_ref, hout1_ref,
               din0_ref, din1_ref, dout0_ref, dout1_ref,
               wga_ref, wgb_ref, bg1_ref, wg2_ref, bg2_ref, out_ref):
    hin = hin0_ref[...] + hin1_ref[...]
    hout = hout0_ref[...] + hout1_ref[...]
    din = jnp.maximum(din0_ref[...] + din1_ref[...], 1.0)
    dout = jnp.maximum(dout0_ref[...] + dout1_ref[...], 1.0)
    h_in = hin / din
    h_out = hout / dout
    gh = jnp.maximum(
        jnp.dot(h_in, wga_ref[...], preferred_element_type=jnp.float32)
        + jnp.dot(h_out, wgb_ref[...], preferred_element_type=jnp.float32)
        + bg1_ref[...], 0.0)
    gz = jnp.sum(gh * wg2_ref[...], axis=1, keepdims=True) + bg2_ref[...]
    g = 1.0 / (1.0 + jnp.exp(-gz))
    out_ref[...] = g * h_in + (1.0 - g) * h_out + x_ref[...]


def _make_edge_kernel(n_nodes, n_edges, d, ck):
    nw = _NC * _NS
    e_per_w = n_edges // nw
    n_chunks = e_per_w // ck
    groups = ck // _L
    assert e_per_w * nw == n_edges and n_chunks * ck == e_per_w
    assert groups * _L == ck and n_nodes % 8 == 0
    span = 8 * (-(-(n_nodes // 8) // _NS))
    zr = 8
    zit = span // zr
    assert zr * zit == span

    mesh = plsc.VectorSubcoreMesh(core_axis_name="c", subcore_axis_name="s",
                                  num_cores=_NC, num_subcores=_NS)

    def body(sd_hbm, a_hbm, b_hbm, p_hbm, q_hbm, wb2_hbm,
             zrow_hbm, zdeg_hbm, ones_hbm,
             hin_out, hout_out, din_out, dout_out, scores_out,
             h_acc, deg_acc,
             idx_a, idx_b, abuf, bbuf, pbuf_a, pbuf_b, sbuf_a, sbuf_b,
             ones_v, zbuf, zdeg, w2_v,
             sem_a, sem_b, sem_p, sem_i, sem_h, sem_g, sem_w):
        c = lax.axis_index("c")
        s = lax.axis_index("s")
        wid = c * _NS + s
        lane = lax.iota(jnp.int32, _L)

        pltpu.sync_copy(wb2_hbm, w2_v)
        pltpu.sync_copy(zrow_hbm, zbuf)
        pltpu.sync_copy(zdeg_hbm, zdeg)
        pltpu.sync_copy(ones_hbm, ones_v)

        r0 = pl.multiple_of(jnp.minimum(s * span, n_nodes - span), 8)

        def clear_acc():
            def zero_step(t, carry):
                off = pl.multiple_of(r0 + t * zr, 8)
                pltpu.sync_copy(zbuf, h_acc.at[pl.ds(off, zr)])
                return carry
            lax.fori_loop(0, zit, zero_step, 0)
            pltpu.sync_copy(zdeg, deg_acc.at[pl.ds(r0, span)])

        clear_acc()
        plsc.subcore_barrier()

        w_slices = [w2_v[pl.ds(j * _L, _L)] for j in range(d // _L)]
        be2s = w2_v[pl.ds(d, _L)][0]
        zero16 = jnp.zeros((_L,), jnp.float32)

        def compute_scores(src_buf1, src_buf2, out_sbuf):
            def group_step(g, carry2):
                e0 = pl.multiple_of(g * _L, _L)
                sv = zero16
                for u in range(_L):
                    e = e0 + u
                    acc = zero16
                    for j in range(d // _L):
                        va = src_buf1[e, pl.ds(j * _L, _L)]
                        vb = src_buf2[e, pl.ds(j * _L, _L)]
                        acc = acc + jnp.maximum(va + vb, 0.0) * w_slices[j]
                    z = jnp.sum(acc) + be2s
                    sv = jnp.where(lane == u, jnp.full((_L,), z, jnp.float32), sv)
                out_sbuf[pl.ds(e0, _L)] = 1.0 / (1.0 + jnp.exp(-sv))
                return carry2

            lax.fori_loop(0, groups, group_step, 0)

        def scale_rows(row_buf, score_buf):
            def group_step(g, carry2):
                e0 = pl.multiple_of(g * _L, _L)
                zv = score_buf[pl.ds(e0, _L)]
                for u in range(_L):
                    e = e0 + u
                    sv = jnp.full((_L,), zv[u], jnp.float32)
                    for j in range(d // _L):
                        sl = pl.ds(j * _L, _L)
                        row_buf[e, sl] = row_buf[e, sl] * sv
                return carry2

            lax.fori_loop(0, groups, group_step, 0)

        def chunk1(i, idx, pb, sb, idxo, pbo, sbo):
            @pl.when(i < n_chunks)
            def _run():
                jid = wid * n_chunks + i

                @pl.when(i > 0)
                def _wait_idx():
                    pltpu.make_async_copy(sd_hbm.at[jid], idx, sem_i).wait()

                cpa = pltpu.async_copy(a_hbm.at[idx.at[0]], abuf, sem_a)
                cpb = pltpu.async_copy(b_hbm.at[idx.at[1]], bbuf, sem_b)
                cpp = pltpu.async_copy(p_hbm.at[idx.at[0]], pb, sem_p)

                @pl.when(i > 0)
                def _drain():
                    pltpu.make_async_copy(pbo, h_acc.at[idxo.at[1]], sem_h).wait()
                    pltpu.make_async_copy(ones_v, deg_acc.at[idxo.at[1]], sem_g).wait()
                    pltpu.make_async_copy(sbo, scores_out.at[pl.ds(0, ck)], sem_w).wait()

                @pl.when(i + 1 < n_chunks)
                def _prefetch():
                    pltpu.async_copy(sd_hbm.at[jid + 1], idxo, sem_i)

                cpa.wait(); cpb.wait()
                compute_scores(abuf, bbuf, sb)
                cpp.wait()
                scale_rows(pb, sb)
                base = pl.multiple_of(jid * ck, 8)
                pltpu.async_copy(pb, h_acc.at[idx.at[1]], sem_h, add=True)
                pltpu.async_copy(ones_v, deg_acc.at[idx.at[1]], sem_g, add=True)
                pltpu.async_copy(sb, scores_out.at[pl.ds(base, ck)], sem_w)

        pltpu.sync_copy(sd_hbm.at[wid * n_chunks], idx_a)

        def pair1(t, carry):
            chunk1(2 * t, idx_a, pbuf_a, sbuf_a, idx_b, pbuf_b, sbuf_b)
            chunk1(2 * t + 1, idx_b, pbuf_b, sbuf_b, idx_a, pbuf_a, sbuf_a)
            return carry

        lax.fori_loop(0, (n_chunks + 1) // 2, pair1, 0)
        lix = idx_a if (n_chunks - 1) % 2 == 0 else idx_b
        lpb = pbuf_a if (n_chunks - 1) % 2 == 0 else pbuf_b
        lsb = sbuf_a if (n_chunks - 1) % 2 == 0 else sbuf_b
        pltpu.make_async_copy(lpb, h_acc.at[lix.at[1]], sem_h).wait()
        pltpu.make_async_copy(ones_v, deg_acc.at[lix.at[1]], sem_g).wait()
        pltpu.make_async_copy(lsb, scores_out.at[pl.ds(0, ck)], sem_w).wait()

        plsc.subcore_barrier()
        out_r0 = pl.multiple_of(c * n_nodes + r0, 8)
        pltpu.sync_copy(h_acc.at[pl.ds(r0, span)],
                        hin_out.at[pl.ds(out_r0, span)])
        pltpu.sync_copy(deg_acc.at[pl.ds(r0, span)],
                        din_out.at[pl.ds(out_r0, span)])
        plsc.subcore_barrier()
        clear_acc()
        plsc.subcore_barrier()

        def chunk2(i, idx, pb, sb, idxo, pbo):
            @pl.when(i < n_chunks)
            def _run():
                jid = wid * n_chunks + i
                base = pl.multiple_of(jid * ck, 8)

                @pl.when(i > 0)
                def _wait_idx():
                    pltpu.make_async_copy(sd_hbm.at[jid], idx, sem_i).wait()

                cpq = pltpu.async_copy(q_hbm.at[idx.at[1]], pb, sem_p)
                cps = pltpu.async_copy(scores_out.at[pl.ds(base, ck)], sb, sem_w)

                @pl.when(i > 0)
                def _drain():
                    pltpu.make_async_copy(pbo, h_acc.at[idxo.at[0]], sem_h).wait()
                    pltpu.make_async_copy(ones_v, deg_acc.at[idxo.at[0]], sem_g).wait()

                @pl.when(i + 1 < n_chunks)
                def _prefetch():
                    pltpu.async_copy(sd_hbm.at[jid + 1], idxo, sem_i)

                cpq.wait(); cps.wait()
                scale_rows(pb, sb)
                pltpu.async_copy(pb, h_acc.at[idx.at[0]], sem_h, add=True)
                pltpu.async_copy(ones_v, deg_acc.at[idx.at[0]], sem_g, add=True)

        pltpu.sync_copy(sd_hbm.at[wid * n_chunks], idx_a)

        def pair2(t, carry):
            chunk2(2 * t, idx_a, pbuf_a, sbuf_a, idx_b, pbuf_b)
            chunk2(2 * t + 1, idx_b, pbuf_b, sbuf_b, idx_a, pbuf_a)
            return carry

        lax.fori_loop(0, (n_chunks + 1) // 2, pair2, 0)
        pltpu.make_async_copy(lpb, h_acc.at[lix.at[0]], sem_h).wait()
        pltpu.make_async_copy(ones_v, deg_acc.at[lix.at[0]], sem_g).wait()

        plsc.subcore_barrier()
        pltpu.sync_copy(h_acc.at[pl.ds(r0, span)],
                        hout_out.at[pl.ds(out_r0, span)])
        pltpu.sync_copy(deg_acc.at[pl.ds(r0, span)],
                        dout_out.at[pl.ds(out_r0, span)])

    return pl.kernel(
        body,
        out_type=[
            jax.ShapeDtypeStruct((_NC * n_nodes, d), jnp.float32),
            jax.ShapeDtypeStruct((_NC * n_nodes, d), jnp.float32),
            jax.ShapeDtypeStruct((_NC * n_nodes,), jnp.float32),
            jax.ShapeDtypeStruct((_NC * n_nodes,), jnp.float32),
            jax.ShapeDtypeStruct((n_edges,), jnp.float32),
        ],
        mesh=mesh,
        scratch_types=[
            pltpu.VMEM_SHARED((n_nodes, d), jnp.float32),
            pltpu.VMEM_SHARED((n_nodes,), jnp.float32),
            pltpu.VMEM((2, ck), jnp.int32),
            pltpu.VMEM((2, ck), jnp.int32),
            pltpu.VMEM((ck, d), jnp.float32),
            pltpu.VMEM((ck, d), jnp.float32),
            pltpu.VMEM((ck, d), jnp.float32),
            pltpu.VMEM((ck, d), jnp.float32),
            pltpu.VMEM((ck,), jnp.float32),
            pltpu.VMEM((ck,), jnp.float32),
            pltpu.VMEM((ck,), jnp.float32),
            pltpu.VMEM((zr, d), jnp.float32),
            pltpu.VMEM((span,), jnp.float32),
            pltpu.VMEM((2 * d,), jnp.float32),
            pltpu.SemaphoreType.DMA,
            pltpu.SemaphoreType.DMA,
            pltpu.SemaphoreType.DMA,
            pltpu.SemaphoreType.DMA,
            pltpu.SemaphoreType.DMA,
            pltpu.SemaphoreType.DMA,
            pltpu.SemaphoreType.DMA,
        ],
        compiler_params=pltpu.CompilerParams(needs_layout_passes=False,
                                             use_tc_tiling_on_sc=False),
    )


def kernel(x, edge_index, W_s2d, b_s2d, W_d2s, b_d2s, We1, be1, We2, be2,
           Wg1, bg1, Wg2, bg2):
    n, d = x.shape
    e = edge_index.shape[1]

    br = 1000 if n % 1000 == 0 else n
    nb = n // br
    row_spec = pl.BlockSpec((br, d), lambda i: (i, 0))
    full_spec = pl.BlockSpec((d, d), lambda i: (0, 0))
    bias_spec = pl.BlockSpec((1, d), lambda i: (0, 0))
    a_n, b_n, p_n, q_n = pl.pallas_call(
        _pre_body,
        grid=(nb,),
        in_specs=[row_spec, full_spec, full_spec, full_spec, full_spec,
                  bias_spec, bias_spec, bias_spec],
        out_specs=[row_spec] * 4,
        out_shape=[jax.ShapeDtypeStruct((n, d), jnp.float32)] * 4,
    )(x, We1[:d], We1[d:], W_s2d, W_d2s,
      be1[None, :], b_s2d[None, :], b_d2s[None, :])

    wb2 = jnp.concatenate([We2[:, 0], be2, jnp.zeros((d - 1,), jnp.float32)])
    ck = 80 if e % (_NC * _NS * 80) == 0 else 16
    total_chunks = e // ck
    edge_sd = edge_index.reshape(2, total_chunks, ck).transpose(1, 0, 2)
    span = 8 * (-(-(n // 8) // _NS))
    zr = 8
    zrow = jnp.zeros((zr, d), jnp.float32)
    zdeg = jnp.zeros((span,), jnp.float32)
    ones = jnp.ones((ck,), jnp.float32)

    edge_fn = _make_edge_kernel(n, e, d, ck)
    hin_p, hout_p, din_p, dout_p, _ = edge_fn(edge_sd, a_n, b_n, p_n, q_n,
                                              wb2, zrow, zdeg, ones)

    lo_spec = pl.BlockSpec((br, d), lambda i: (i, 0))
    hi_spec = pl.BlockSpec((br, d), lambda i: (i + nb, 0))
    dlo_spec = pl.BlockSpec((br, 1), lambda i: (i, 0))
    dhi_spec = pl.BlockSpec((br, 1), lambda i: (i + nb, 0))
    din2 = din_p[:, None]
    dout2 = dout_p[:, None]
    out = pl.pallas_call(
        _post_body,
        grid=(nb,),
        in_specs=[row_spec, lo_spec, hi_spec, lo_spec, hi_spec,
                  dlo_spec, dhi_spec, dlo_spec, dhi_spec,
                  full_spec, full_spec, bias_spec, bias_spec,
                  pl.BlockSpec((1, 1), lambda i: (0, 0))],
        out_specs=row_spec,
        out_shape=jax.ShapeDtypeStruct((n, d), jnp.float32),
    )(x, hin_p, hin_p, hout_p, hout_p, din2, din2, dout2, dout2,
      Wg1[:d], Wg1[d:], bg1[None, :], Wg2[:, 0][None, :], bg2[:, None])
    return out


# speedup vs baseline: 1.3388x; 1.3388x over previous
import jax
import jax.numpy as jnp
from jax import lax
from jax.experimental import pallas as pl
from jax.experimental.pallas import tpu as pltpu
from jax.experimental.pallas import tpu_sc as plsc

_NC = 2
_NS = 16
_L = 16

def _pre_body(x_ref, wa_ref, wb_ref, wp_ref, wq_ref, bb_ref, bp_ref, bq_ref,
              a_out, b_out, p_out, q_out):
    x = x_ref[...]
    a_out[...] = jnp.dot(x, wa_ref[...], preferred_element_type=jnp.float32).astype(jnp.bfloat16)
    b_out[...] = (jnp.dot(x, wb_ref[...], preferred_element_type=jnp.float32) + bb_ref[...]).astype(jnp.bfloat16)
    p_out[...] = jnp.dot(x, wp_ref[...], preferred_element_type=jnp.float32) + bp_ref[...]
    q_out[...] = jnp.dot(x, wq_ref[...], preferred_element_type=jnp.float32) + bq_ref[...]


def _post_body(x_ref, hin0_ref, hin1_ref, hout0_ref, hout1_ref,
               din0_ref, din1_ref, dout0_ref, dout1_ref,
               wga_ref, wgb_ref, bg1_ref, wg2_ref, bg2_ref, out_ref):
    hin = hin0_ref[...] + hin1_ref[...]
    hout = hout0_ref[...] + hout1_ref[...]
    din = jnp.maximum(din0_ref[...] + din1_ref[...], 1.0)
    dout = jnp.maximum(dout0_ref[...] + dout1_ref[...], 1.0)
    h_in = hin / din
    h_out = hout / dout
    gh = jnp.maximum(
        jnp.dot(h_in, wga_ref[...], preferred_element_type=jnp.float32)
        + jnp.dot(h_out, wgb_ref[...], preferred_element_type=jnp.float32)
        + bg1_ref[...], 0.0)
    gz = jnp.sum(gh * wg2_ref[...], axis=1, keepdims=True) + bg2_ref[...]
    g = 1.0 / (1.0 + jnp.exp(-gz))
    out_ref[...] = g * h_in + (1.0 - g) * h_out + x_ref[...]


def _make_edge_kernel(n_nodes, n_edges, d, ck):
    nw = _NC * _NS
    e_per_w = n_edges // nw
    n_chunks = e_per_w // ck
    groups = ck // _L
    assert e_per_w * nw == n_edges and n_chunks * ck == e_per_w
    assert groups * _L == ck and n_nodes % 8 == 0
    span = 8 * (-(-(n_nodes // 8) // _NS))
    zr = 8
    zit = span // zr
    assert zr * zit == span

    mesh = plsc.VectorSubcoreMesh(core_axis_name="c", subcore_axis_name="s",
                                  num_cores=_NC, num_subcores=_NS)

    def body(sd_hbm, a_hbm, b_hbm, p_hbm, q_hbm, wb2_hbm, be2_hbm,
             zrow_hbm, zdeg_hbm, ones_hbm,
             hin_out, hout_out, din_out, dout_out, scores_out,
             h_acc, deg_acc,
             idx_a, idx_b, abuf, bbuf, pbuf_a, pbuf_b, sbuf_a, sbuf_b,
             ones_v, zbuf, zdeg, w2_v, be2_v,
             sem_a, sem_b, sem_p, sem_i, sem_h, sem_g, sem_w):
        c = lax.axis_index("c")
        s = lax.axis_index("s")
        wid = c * _NS + s
        lane = lax.iota(jnp.int32, _L)

        pltpu.sync_copy(wb2_hbm, w2_v)
        pltpu.sync_copy(be2_hbm, be2_v)
        pltpu.sync_copy(zrow_hbm, zbuf)
        pltpu.sync_copy(zdeg_hbm, zdeg)
        pltpu.sync_copy(ones_hbm, ones_v)

        r0 = pl.multiple_of(jnp.minimum(s * span, n_nodes - span), 8)

        def clear_acc():
            def zero_step(t, carry):
                off = pl.multiple_of(r0 + t * zr, 8)
                pltpu.sync_copy(zbuf, h_acc.at[pl.ds(off, zr)])
                return carry
            lax.fori_loop(0, zit, zero_step, 0)
            pltpu.sync_copy(zdeg, deg_acc.at[pl.ds(r0, span)])

        clear_acc()
        plsc.subcore_barrier()

        w_pairs = [plsc.unpack(w2_v[pl.ds(j * 2 * _L, 2 * _L)],
                               format=plsc.PackFormat.INTERLEAVED)
                   for j in range(d // (2 * _L))]
        be2s = be2_v[pl.ds(0, _L)][0]
        zero16 = jnp.zeros((_L,), jnp.float32)
        zbf = jnp.zeros((2 * _L,), jnp.bfloat16)

        def compute_scores(src_buf1, src_buf2, out_sbuf):
            def group_step(g, carry2):
                e0 = pl.multiple_of(g * _L, _L)
                sv = zero16
                for u in range(_L):
                    e = e0 + u
                    acc = zero16
                    for j in range(d // (2 * _L)):
                        sl = pl.ds(j * 2 * _L, 2 * _L)
                        t = jnp.maximum(src_buf1[e, sl] + src_buf2[e, sl], zbf)
                        tu, to = plsc.unpack(t, format=plsc.PackFormat.INTERLEAVED)
                        wu, wo = w_pairs[j]
                        acc = acc + tu * wu + to * wo
                    z = jnp.sum(acc) + be2s
                    sv = jnp.where(lane == u, jnp.full((_L,), z, jnp.float32), sv)
                out_sbuf[pl.ds(e0, _L)] = 1.0 / (1.0 + jnp.exp(-sv))
                return carry2

            lax.fori_loop(0, groups, group_step, 0)

        def scale_rows(row_buf, score_buf):
            def group_step(g, carry2):
                e0 = pl.multiple_of(g * _L, _L)
                zv = score_buf[pl.ds(e0, _L)]
                for u in range(_L):
                    e = e0 + u
                    sv = jnp.full((_L,), zv[u], jnp.float32)
                    for j in range(d // _L):
                        sl = pl.ds(j * _L, _L)
                        row_buf[e, sl] = row_buf[e, sl] * sv
                return carry2

            lax.fori_loop(0, groups, group_step, 0)

        def chunk1(i, idx, pb, sb, idxo, pbo, sbo):
            @pl.when(i < n_chunks)
            def _run():
                jid = wid * n_chunks + i

                @pl.when(i > 0)
                def _wait_idx():
                    pltpu.make_async_copy(sd_hbm.at[jid], idx, sem_i).wait()

                cpa = pltpu.async_copy(a_hbm.at[idx.at[0]], abuf, sem_a)
                cpb = pltpu.async_copy(b_hbm.at[idx.at[1]], bbuf, sem_b)
                cpp = pltpu.async_copy(p_hbm.at[idx.at[0]], pb, sem_p)

                @pl.when(i > 0)
                def _drain():
                    pltpu.make_async_copy(pbo, h_acc.at[idxo.at[1]], sem_h).wait()
                    pltpu.make_async_copy(ones_v, deg_acc.at[idxo.at[1]], sem_g).wait()
                    pltpu.make_async_copy(sbo, scores_out.at[pl.ds(0, ck)], sem_w).wait()

                @pl.when(i + 1 < n_chunks)
                def _prefetch():
                    pltpu.async_copy(sd_hbm.at[jid + 1], idxo, sem_i)

                cpa.wait(); cpb.wait()
                compute_scores(abuf, bbuf, sb)
                cpp.wait()
                scale_rows(pb, sb)
                base = pl.multiple_of(jid * ck, 8)
                pltpu.async_copy(pb, h_acc.at[idx.at[1]], sem_h, add=True)
                pltpu.async_copy(ones_v, deg_acc.at[idx.at[1]], sem_g, add=True)
                pltpu.async_copy(sb, scores_out.at[pl.ds(base, ck)], sem_w)

        pltpu.sync_copy(sd_hbm.at[wid * n_chunks], idx_a)

        def pair1(t, carry):
            chunk1(2 * t, idx_a, pbuf_a, sbuf_a, idx_b, pbuf_b, sbuf_b)
            chunk1(2 * t + 1, idx_b, pbuf_b, sbuf_b, idx_a, pbuf_a, sbuf_a)
            return carry

        lax.fori_loop(0, (n_chunks + 1) // 2, pair1, 0)
        lix = idx_a if (n_chunks - 1) % 2 == 0 else idx_b
        lpb = pbuf_a if (n_chunks - 1) % 2 == 0 else pbuf_b
        lsb = sbuf_a if (n_chunks - 1) % 2 == 0 else sbuf_b
        pltpu.make_async_copy(lpb, h_acc.at[lix.at[1]], sem_h).wait()
        pltpu.make_async_copy(ones_v, deg_acc.at[lix.at[1]], sem_g).wait()
        pltpu.make_async_copy(lsb, scores_out.at[pl.ds(0, ck)], sem_w).wait()

        plsc.subcore_barrier()
        out_r0 = pl.multiple_of(c * n_nodes + r0, 8)
        pltpu.sync_copy(h_acc.at[pl.ds(r0, span)],
                        hin_out.at[pl.ds(out_r0, span)])
        pltpu.sync_copy(deg_acc.at[pl.ds(r0, span)],
                        din_out.at[pl.ds(out_r0, span)])
        plsc.subcore_barrier()
        clear_acc()
        plsc.subcore_barrier()

        def chunk2(i, idx, pb, sb, idxo, pbo):
            @pl.when(i < n_chunks)
            def _run():
                jid = wid * n_chunks + i
                base = pl.multiple_of(jid * ck, 8)

                @pl.when(i > 0)
                def _wait_idx():
                    pltpu.make_async_copy(sd_hbm.at[jid], idx, sem_i).wait()

                cpq = pltpu.async_copy(q_hbm.at[idx.at[1]], pb, sem_p)
                cps = pltpu.async_copy(scores_out.at[pl.ds(base, ck)], sb, sem_w)

                @pl.when(i > 0)
                def _drain():
                    pltpu.make_async_copy(pbo, h_acc.at[idxo.at[0]], sem_h).wait()
                    pltpu.make_async_copy(ones_v, deg_acc.at[idxo.at[0]], sem_g).wait()

                @pl.when(i + 1 < n_chunks)
                def _prefetch():
                    pltpu.async_copy(sd_hbm.at[jid + 1], idxo, sem_i)

                cpq.wait(); cps.wait()
                scale_rows(pb, sb)
                pltpu.async_copy(pb, h_acc.at[idx.at[0]], sem_h, add=True)
                pltpu.async_copy(ones_v, deg_acc.at[idx.at[0]], sem_g, add=True)

        pltpu.sync_copy(sd_hbm.at[wid * n_chunks], idx_a)

        def pair2(t, carry):
            chunk2(2 * t, idx_a, pbuf_a, sbuf_a, idx_b, pbuf_b)
            chunk2(2 * t + 1, idx_b, pbuf_b, sbuf_b, idx_a, pbuf_a)
            return carry

        lax.fori_loop(0, (n_chunks + 1) // 2, pair2, 0)
        pltpu.make_async_copy(lpb, h_acc.at[lix.at[0]], sem_h).wait()
        pltpu.make_async_copy(ones_v, deg_acc.at[lix.at[0]], sem_g).wait()

        plsc.subcore_barrier()
        pltpu.sync_copy(h_acc.at[pl.ds(r0, span)],
                        hout_out.at[pl.ds(out_r0, span)])
        pltpu.sync_copy(deg_acc.at[pl.ds(r0, span)],
                        dout_out.at[pl.ds(out_r0, span)])

    return pl.kernel(
        body,
        out_type=[
            jax.ShapeDtypeStruct((_NC * n_nodes, d), jnp.float32),
            jax.ShapeDtypeStruct((_NC * n_nodes, d), jnp.float32),
            jax.ShapeDtypeStruct((_NC * n_nodes,), jnp.float32),
            jax.ShapeDtypeStruct((_NC * n_nodes,), jnp.float32),
            jax.ShapeDtypeStruct((n_edges,), jnp.float32),
        ],
        mesh=mesh,
        scratch_types=[
            pltpu.VMEM_SHARED((n_nodes, d), jnp.float32),
            pltpu.VMEM_SHARED((n_nodes,), jnp.float32),
            pltpu.VMEM((2, ck), jnp.int32),
            pltpu.VMEM((2, ck), jnp.int32),
            pltpu.VMEM((ck, d), jnp.bfloat16),
            pltpu.VMEM((ck, d), jnp.bfloat16),
            pltpu.VMEM((ck, d), jnp.float32),
            pltpu.VMEM((ck, d), jnp.float32),
            pltpu.VMEM((ck,), jnp.float32),
            pltpu.VMEM((ck,), jnp.float32),
            pltpu.VMEM((ck,), jnp.float32),
            pltpu.VMEM((zr, d), jnp.float32),
            pltpu.VMEM((span,), jnp.float32),
            pltpu.VMEM((2 * d,), jnp.bfloat16),
            pltpu.VMEM((_L,), jnp.float32),
            pltpu.SemaphoreType.DMA,
            pltpu.SemaphoreType.DMA,
            pltpu.SemaphoreType.DMA,
            pltpu.SemaphoreType.DMA,
            pltpu.SemaphoreType.DMA,
            pltpu.SemaphoreType.DMA,
            pltpu.SemaphoreType.DMA,
        ],
        compiler_params=pltpu.CompilerParams(needs_layout_passes=False,
                                             use_tc_tiling_on_sc=False),
    )


def kernel(x, edge_index, W_s2d, b_s2d, W_d2s, b_d2s, We1, be1, We2, be2,
           Wg1, bg1, Wg2, bg2):
    n, d = x.shape
    e = edge_index.shape[1]

    br = 1000 if n % 1000 == 0 else n
    nb = n // br
    row_spec = pl.BlockSpec((br, d), lambda i: (i, 0))
    full_spec = pl.BlockSpec((d, d), lambda i: (0, 0))
    bias_spec = pl.BlockSpec((1, d), lambda i: (0, 0))
    a_n, b_n, p_n, q_n = pl.pallas_call(
        _pre_body,
        grid=(nb,),
        in_specs=[row_spec, full_spec, full_spec, full_spec, full_spec,
                  bias_spec, bias_spec, bias_spec],
        out_specs=[row_spec] * 4,
        out_shape=[jax.ShapeDtypeStruct((n, d), jnp.bfloat16)] * 2
        + [jax.ShapeDtypeStruct((n, d), jnp.float32)] * 2,
    )(x, We1[:d], We1[d:], W_s2d, W_d2s,
      be1[None, :], b_s2d[None, :], b_d2s[None, :])

    wb2 = jnp.pad(We2[:, 0], (0, d)).astype(jnp.bfloat16)
    be2p = jnp.full((_L,), be2[0], jnp.float32)
    ck = 80 if e % (_NC * _NS * 80) == 0 else 16
    total_chunks = e // ck
    edge_sd = edge_index.reshape(2, total_chunks, ck).transpose(1, 0, 2)
    span = 8 * (-(-(n // 8) // _NS))
    zr = 8
    zrow = jnp.zeros((zr, d), jnp.float32)
    zdeg = jnp.zeros((span,), jnp.float32)
    ones = jnp.ones((ck,), jnp.float32)

    edge_fn = _make_edge_kernel(n, e, d, ck)
    hin_p, hout_p, din_p, dout_p, _ = edge_fn(edge_sd, a_n, b_n, p_n, q_n,
                                              wb2, be2p, zrow, zdeg, ones)

    lo_spec = pl.BlockSpec((br, d), lambda i: (i, 0))
    hi_spec = pl.BlockSpec((br, d), lambda i: (i + nb, 0))
    dlo_spec = pl.BlockSpec((br, 1), lambda i: (i, 0))
    dhi_spec = pl.BlockSpec((br, 1), lambda i: (i + nb, 0))
    din2 = din_p[:, None]
    dout2 = dout_p[:, None]
    out = pl.pallas_call(
        _post_body,
        grid=(nb,),
        in_specs=[row_spec, lo_spec, hi_spec, lo_spec, hi_spec,
                  dlo_spec, dhi_spec, dlo_spec, dhi_spec,
                  full_spec, full_spec, bias_spec, bias_spec,
                  pl.BlockSpec((1, 1), lambda i: (0, 0))],
        out_specs=row_spec,
        out_shape=jax.ShapeDtypeStruct((n, d), jnp.float32),
    )(x, hin_p, hin_p, hout_p, hout_p, din2, din2, dout2, dout2,
      Wg1[:d], Wg1[d:], bg1[None, :], Wg2[:, 0][None, :], bg2[:, None])
    return out
